# baseline (speedup 1.0000x reference)
import functools

import jax
import jax.numpy as jnp
from jax import lax
from jax.experimental import pallas as pl
from jax.experimental.pallas import tpu as pltpu
from jax.experimental.pallas import tpu_sc as plsc

N = 10000
NP = 10240
E = 320000
D = 128
CH = 128
ECH = 2560
EP = ECH * CH
BR = 512
GRID = NP // BR
RPT = NP // 16
SLOPE = (1.0 / 8.0 + 1.0 / 3.0) / 2.0

NJ = ECH // 16
BT = 16
NB = NJ // BT
NRING = 2
NJD = ECH // 32


def _mesh():
    return plsc.VectorSubcoreMesh(core_axis_name="c", subcore_axis_name="s")


def _sc_deg_body(dst_hbm, out_hbm, didx, ones_v, zbuf, acc):
    cid = lax.axis_index("c")
    sid = lax.axis_index("s")
    wid = cid * 16 + sid

    for l in range(8):
        ones_v[pl.ds(l * 16, 16)] = jnp.full((16,), 1.0, jnp.float32)
    zeros16 = jnp.zeros((16,), jnp.float32)

    @pl.loop(0, RPT // 16)
    def _zero(i):
        zbuf[pl.ds(i * 16, 16)] = zeros16

    pltpu.sync_copy(zbuf, acc.at[pl.ds(sid * RPT, RPT)])
    plsc.subcore_barrier()

    @pl.loop(0, NJD)
    def _edges(j):
        off = (wid + 32 * j) * CH
        pltpu.sync_copy(dst_hbm.at[pl.ds(off, CH)], didx)
        pltpu.sync_copy(ones_v, acc.at[didx], add=True)

    plsc.subcore_barrier()
    pltpu.sync_copy(acc.at[pl.ds(sid * RPT, RPT)],
                    out_hbm.at[cid, pl.ds(sid * RPT, RPT)])


def _sc_deg(dst):
    f = functools.partial(
        pl.kernel,
        out_type=jax.ShapeDtypeStruct((2, NP), jnp.float32),
        mesh=_mesh(),
        scratch_types=[
            pltpu.VMEM((CH,), jnp.int32),
            pltpu.VMEM((CH,), jnp.float32),
            pltpu.VMEM((RPT,), jnp.float32),
            pltpu.VMEM_SHARED((NP,), jnp.float32),
        ],
    )(_sc_deg_body)
    return f(dst)


def _sc_agg_body(g_hbm, src_hbm, dst_hbm, out_hbm,
                 sidxA, didxA, sidxB, didxB,
                 rows0, rows1,
                 acc, isem, gsem0, gsem1, ssem0, ssem1):
    cid = lax.axis_index("c")
    sid = lax.axis_index("s")
    gv = g_hbm.at[cid]

    sbuf = (sidxA, sidxB)
    dbuf = (didxA, didxB)
    rbuf = (rows0, rows1)
    gsem = (gsem0, gsem1)
    ssem = (ssem0, ssem1)

    zeros16 = jnp.zeros((16,), jnp.float32)

    @pl.loop(0, CH)
    def _zrow(r):
        for l in range(D // 16):
            rows0[r, pl.ds(l * 16, 16)] = zeros16

    for k in range(RPT // CH):
        pltpu.sync_copy(rows0, acc.at[pl.ds(sid * RPT + k * CH, CH)])
    plsc.subcore_barrier()

    cbase = sid * NJ

    def load_idx_batch(t, ib):
        row = cbase + t * BT
        pltpu.async_copy(src_hbm.at[pl.ds(row, BT)], sbuf[ib], isem)
        pltpu.async_copy(dst_hbm.at[pl.ds(row, BT)], dbuf[ib], isem)

    def wait_idx_batch(ib):
        pltpu.make_async_copy(src_hbm.at[pl.ds(0, BT)], sbuf[ib], isem).wait()
        pltpu.make_async_copy(dst_hbm.at[pl.ds(0, BT)], dbuf[ib], isem).wait()

    def start_gather(ib, k, b):
        pltpu.async_copy(gv.at[sbuf[ib].at[k]], rbuf[b], gsem[b])

    def wait_gather(ib, k, b):
        pltpu.make_async_copy(gv.at[sbuf[ib].at[k]], rbuf[b], gsem[b]).wait()

    def start_scatter(ib, k, b):
        pltpu.async_copy(rbuf[b], acc.at[dbuf[ib].at[k]], ssem[b], add=True)

    def wait_scatter(ib, k, b):
        pltpu.make_async_copy(rbuf[b], acc.at[dbuf[ib].at[k]], ssem[b]).wait()

    load_idx_batch(0, 0)
    wait_idx_batch(0)
    load_idx_batch(1, 1)
    for k in range(BT):
        b = k % NRING
        if k >= NRING:
            wait_scatter(0, k - NRING, b)
        start_gather(0, k, b)
        if k >= 1:
            bp = (k - 1) % NRING
            wait_gather(0, k - 1, bp)
            start_scatter(0, k - 1, bp)

    @pl.loop(1, NB)
    def _batch(t):
        tb = t % 2
        for parity in range(2):
            @pl.when(tb == parity)
            def _run(parity=parity):
                ib = parity
                nib = 1 - parity
                for k in range(BT):
                    b = k % NRING
                    if k == 0:
                        wait_gather(nib, BT - 1, (BT - 1) % NRING)
                        start_scatter(nib, BT - 1, (BT - 1) % NRING)
                        wait_idx_batch(ib)
                    if k == 2:
                        @pl.when(t + 1 < NB)
                        def _pf():
                            load_idx_batch(t + 1, nib)
                    wait_scatter(ib, k - NRING, b)
                    start_gather(ib, k, b)
                    if k >= 1:
                        bp = (k - 1) % NRING
                        wait_gather(ib, k - 1, bp)
                        start_scatter(ib, k - 1, bp)

    fib = (NB - 1) % 2
    kk = BT - 1
    wait_gather(fib, kk, kk % NRING)
    start_scatter(fib, kk, kk % NRING)
    wait_scatter(fib, BT - 2, (BT - 2) % NRING)
    wait_scatter(fib, BT - 1, (BT - 1) % NRING)

    plsc.subcore_barrier()
    for k in range(RPT // CH):
        pltpu.sync_copy(acc.at[pl.ds(sid * RPT + k * CH, CH)],
                        out_hbm.at[cid].at[pl.ds(sid * RPT + k * CH, CH)])


def _sc_agg(g, src2d, dst2d):
    f = functools.partial(
        pl.kernel,
        out_type=jax.ShapeDtypeStruct((2, NP, D), jnp.float32),
        mesh=_mesh(),
        scratch_types=[
            pltpu.VMEM((BT, CH), jnp.int32),
            pltpu.VMEM((BT, CH), jnp.int32),
            pltpu.VMEM((BT, CH), jnp.int32),
            pltpu.VMEM((BT, CH), jnp.int32),
            pltpu.VMEM((CH, D), jnp.float32),
            pltpu.VMEM((CH, D), jnp.float32),
            pltpu.VMEM_SHARED((NP, D), jnp.float32),
            pltpu.SemaphoreType.DMA,
            pltpu.SemaphoreType.DMA,
            pltpu.SemaphoreType.DMA,
            pltpu.SemaphoreType.DMA,
            pltpu.SemaphoreType.DMA,
        ],
    )(_sc_agg_body)
    return f(g, src2d, dst2d)


def _tc1_body(z_ref, x_ref, d2_ref, w_ref, g_ref, dinv_ref):
    deg = d2_ref[0] + d2_ref[1] + 1.0
    dinv = lax.rsqrt(deg)
    dinv_ref[...] = dinv
    g_ref[0] = dinv * jnp.dot(z_ref[...], w_ref[0],
                              preferred_element_type=jnp.float32)
    g_ref[1] = dinv * jnp.dot(x_ref[...], w_ref[1],
                              preferred_element_type=jnp.float32)


def _tc1(z_pad, x_pad, deg2, w1):
    return pl.pallas_call(
        _tc1_body,
        grid=(GRID,),
        in_specs=[
            pl.BlockSpec((BR, D), lambda i: (i, 0)),
            pl.BlockSpec((BR, D), lambda i: (i, 0)),
            pl.BlockSpec((2, BR, 1), lambda i: (0, i, 0)),
            pl.BlockSpec((2, D, D), lambda i: (0, 0, 0)),
        ],
        out_specs=[
            pl.BlockSpec((2, BR, D), lambda i: (0, i, 0)),
            pl.BlockSpec((BR, 1), lambda i: (i, 0)),
        ],
        out_shape=[
            jax.ShapeDtypeStruct((2, NP, D), jnp.float32),
            jax.ShapeDtypeStruct((NP, 1), jnp.float32),
        ],
    )(z_pad, x_pad, deg2, w1)


def _tc2_body(agg_ref, g_ref, dinv_ref, b_ref, w_ref, out_ref):
    dinv = dinv_ref[...]
    for c in range(2):
        u = dinv * (agg_ref[c] + g_ref[c]) + b_ref[c]
        u = jnp.where(u >= 0, u, u * SLOPE)
        out_ref[c] = dinv * jnp.dot(u, w_ref[c],
                                    preferred_element_type=jnp.float32)


def _tc2(agg1, g1, dinv, b1, w2):
    return pl.pallas_call(
        _tc2_body,
        grid=(GRID,),
        in_specs=[
            pl.BlockSpec((2, BR, D), lambda i: (0, i, 0)),
            pl.BlockSpec((2, BR, D), lambda i: (0, i, 0)),
            pl.BlockSpec((BR, 1), lambda i: (i, 0)),
            pl.BlockSpec((2, D), lambda i: (0, 0)),
            pl.BlockSpec((2, D, D), lambda i: (0, 0, 0)),
        ],
        out_specs=pl.BlockSpec((2, BR, D), lambda i: (0, i, 0)),
        out_shape=jax.ShapeDtypeStruct((2, NP, D), jnp.float32),
    )(agg1, g1, dinv, b1, w2)


def _tc3_body(agg_ref, g_ref, dinv_ref, b_ref, wo_ref, bo_ref, out_ref):
    dinv = dinv_ref[...]
    zz = jnp.tanh(dinv * (agg_ref[0] + g_ref[0]) + b_ref[0])
    xx = jnp.tanh(dinv * (agg_ref[1] + g_ref[1]) + b_ref[1])
    out_ref[...] = jnp.dot(zz * xx, wo_ref[...],
                           preferred_element_type=jnp.float32) + bo_ref[...]


def _tc3(agg2, g2, dinv, b2, Wo, bo):
    return pl.pallas_call(
        _tc3_body,
        grid=(GRID,),
        in_specs=[
            pl.BlockSpec((2, BR, D), lambda i: (0, i, 0)),
            pl.BlockSpec((2, BR, D), lambda i: (0, i, 0)),
            pl.BlockSpec((BR, 1), lambda i: (i, 0)),
            pl.BlockSpec((2, D), lambda i: (0, 0)),
            pl.BlockSpec((D, 1), lambda i: (0, 0)),
            pl.BlockSpec((1,), lambda i: (0,)),
        ],
        out_specs=pl.BlockSpec((BR, 1), lambda i: (i, 0)),
        out_shape=jax.ShapeDtypeStruct((NP, 1), jnp.float32),
    )(agg2, g2, dinv, b2, Wo, bo)


@jax.jit
def kernel(z, x, edge_index, We1, be1, We2, be2, Wf1, bf1, Wf2, bf2, Wo, bo):
    src_p = jnp.concatenate(
        [edge_index[0], jnp.zeros((EP - E,), jnp.int32)]).reshape(ECH, CH)
    dst_p = jnp.concatenate(
        [edge_index[1], jnp.full((EP - E,), N, jnp.int32)]).reshape(ECH, CH)

    z_pad = jnp.pad(z, ((0, NP - N), (0, 0)))
    x_pad = jnp.pad(x, ((0, NP - N), (0, 0)))
    w1 = jnp.stack([We1, Wf1])
    w2 = jnp.stack([We2, Wf2])
    b1 = jnp.stack([be1, bf1])
    b2 = jnp.stack([be2, bf2])

    deg2 = _sc_deg(dst_p.reshape(EP))
    deg2 = deg2[:, :, None]

    g1, dinv = _tc1(z_pad, x_pad, deg2, w1)
    agg1 = _sc_agg(g1, src_p, dst_p)
    g2 = _tc2(agg1, g1, dinv, b1, w2)
    agg2 = _sc_agg(g2, src_p, dst_p)
    out = _tc3(agg2, g2, dinv, b2, Wo, bo)
    return out[:N]


# speedup vs baseline: 1.0030x; 1.0030x over previous
import functools

import jax
import jax.numpy as jnp
from jax import lax
from jax.experimental import pallas as pl
from jax.experimental.pallas import tpu as pltpu
from jax.experimental.pallas import tpu_sc as plsc

N = 10000
NP = 10240
E = 320000
D = 128
CH = 128
ECH = 2560
EP = ECH * CH
BR = 512
GRID = NP // BR
RPT = NP // 16
SLOPE = (1.0 / 8.0 + 1.0 / 3.0) / 2.0

NJ = ECH // 16
BT = 16
NB = NJ // BT
NRING = 2
NJD = ECH // 32


def _mesh():
    return plsc.VectorSubcoreMesh(core_axis_name="c", subcore_axis_name="s")


def _sc_deg_body(dst_hbm, out_hbm, didx, ones_v, zbuf, acc):
    cid = lax.axis_index("c")
    sid = lax.axis_index("s")
    wid = cid * 16 + sid

    for l in range(8):
        ones_v[pl.ds(l * 16, 16)] = jnp.full((16,), 1.0, jnp.float32)
    zeros16 = jnp.zeros((16,), jnp.float32)

    @pl.loop(0, RPT // 16)
    def _zero(i):
        zbuf[pl.ds(i * 16, 16)] = zeros16

    pltpu.sync_copy(zbuf, acc.at[pl.ds(sid * RPT, RPT)])
    plsc.subcore_barrier()

    @pl.loop(0, NJD)
    def _edges(j):
        off = (wid + 32 * j) * CH
        pltpu.sync_copy(dst_hbm.at[pl.ds(off, CH)], didx)
        pltpu.sync_copy(ones_v, acc.at[didx], add=True)

    plsc.subcore_barrier()
    pltpu.sync_copy(acc.at[pl.ds(sid * RPT, RPT)],
                    out_hbm.at[cid, pl.ds(sid * RPT, RPT)])


def _sc_deg(dst):
    f = functools.partial(
        pl.kernel,
        out_type=jax.ShapeDtypeStruct((2, NP), jnp.float32),
        mesh=_mesh(),
        scratch_types=[
            pltpu.VMEM((CH,), jnp.int32),
            pltpu.VMEM((CH,), jnp.float32),
            pltpu.VMEM((RPT,), jnp.float32),
            pltpu.VMEM_SHARED((NP,), jnp.float32),
        ],
    )(_sc_deg_body)
    return f(dst)


def _sc_agg_body(g_hbm, src_hbm, dst_hbm, out_hbm,
                 sidxA, didxA, sidxB, didxB,
                 rows0, rows1,
                 acc, isem, gsem0, gsem1):
    cid = lax.axis_index("c")
    sid = lax.axis_index("s")
    gv = g_hbm.at[cid]

    sbuf = (sidxA, sidxB)
    dbuf = (didxA, didxB)
    rbuf = (rows0, rows1)
    gsem = (gsem0, gsem1)

    zeros16 = jnp.zeros((16,), jnp.float32)

    @pl.loop(0, CH)
    def _zrow(r):
        for l in range(D // 16):
            rows0[r, pl.ds(l * 16, 16)] = zeros16

    for k in range(RPT // CH):
        pltpu.sync_copy(rows0, acc.at[pl.ds(sid * RPT + k * CH, CH)])
    plsc.subcore_barrier()

    cbase = sid * NJ

    def load_idx_batch(t, ib):
        row = cbase + t * BT
        pltpu.async_copy(src_hbm.at[pl.ds(row, BT)], sbuf[ib], isem)
        pltpu.async_copy(dst_hbm.at[pl.ds(row, BT)], dbuf[ib], isem)

    def wait_idx_batch(ib):
        pltpu.make_async_copy(src_hbm.at[pl.ds(0, BT)], sbuf[ib], isem).wait()
        pltpu.make_async_copy(dst_hbm.at[pl.ds(0, BT)], dbuf[ib], isem).wait()

    def start_gather(ib, k, b):
        pltpu.async_copy(gv.at[sbuf[ib].at[k]], rbuf[b], gsem[b])

    def wait_gather(ib, k, b):
        pltpu.make_async_copy(gv.at[sbuf[ib].at[k]], rbuf[b], gsem[b]).wait()

    def scatter(ib, k, b):
        pltpu.sync_copy(rbuf[b], acc.at[dbuf[ib].at[k]], add=True)

    load_idx_batch(0, 0)
    wait_idx_batch(0)
    start_gather(0, 0, 0)
    for k in range(BT):
        b = k % NRING
        nb_ = (k + 1) % NRING
        if k == 2:
            load_idx_batch(1, 1)
        if k == BT - 2:
            wait_idx_batch(1)
        if k < BT - 1:
            start_gather(0, k + 1, nb_)
        else:
            start_gather(1, 0, nb_)
        wait_gather(0, k, b)
        scatter(0, k, b)

    @pl.loop(1, NB)
    def _batch(t):
        tb = t % 2
        for parity in range(2):
            @pl.when(tb == parity)
            def _run(parity=parity):
                ib = parity
                nib = 1 - parity
                for k in range(BT):
                    b = k % NRING
                    nb_ = (k + 1) % NRING
                    if k == 2:
                        @pl.when(t + 1 < NB)
                        def _pf():
                            load_idx_batch(t + 1, nib)
                    if k == BT - 2:
                        @pl.when(t + 1 < NB)
                        def _wf():
                            wait_idx_batch(nib)
                    if k < BT - 1:
                        start_gather(ib, k + 1, nb_)
                    else:
                        @pl.when(t + 1 < NB)
                        def _ng():
                            start_gather(nib, 0, nb_)
                    wait_gather(ib, k, b)
                    scatter(ib, k, b)

    plsc.subcore_barrier()
    for k in range(RPT // CH):
        pltpu.sync_copy(acc.at[pl.ds(sid * RPT + k * CH, CH)],
                        out_hbm.at[cid].at[pl.ds(sid * RPT + k * CH, CH)])


def _sc_agg(g, src2d, dst2d):
    f = functools.partial(
        pl.kernel,
        out_type=jax.ShapeDtypeStruct((2, NP, D), jnp.float32),
        mesh=_mesh(),
        scratch_types=[
            pltpu.VMEM((BT, CH), jnp.int32),
            pltpu.VMEM((BT, CH), jnp.int32),
            pltpu.VMEM((BT, CH), jnp.int32),
            pltpu.VMEM((BT, CH), jnp.int32),
            pltpu.VMEM((CH, D), jnp.float32),
            pltpu.VMEM((CH, D), jnp.float32),
            pltpu.VMEM_SHARED((NP, D), jnp.float32),
            pltpu.SemaphoreType.DMA,
            pltpu.SemaphoreType.DMA,
            pltpu.SemaphoreType.DMA,
        ],
    )(_sc_agg_body)
    return f(g, src2d, dst2d)


def _tc1_body(z_ref, x_ref, d2_ref, w_ref, g_ref, dinv_ref):
    deg = d2_ref[0] + d2_ref[1] + 1.0
    dinv = lax.rsqrt(deg)
    dinv_ref[...] = dinv
    g_ref[0] = dinv * jnp.dot(z_ref[...], w_ref[0],
                              preferred_element_type=jnp.float32)
    g_ref[1] = dinv * jnp.dot(x_ref[...], w_ref[1],
                              preferred_element_type=jnp.float32)


def _tc1(z_pad, x_pad, deg2, w1):
    return pl.pallas_call(
        _tc1_body,
        grid=(GRID,),
        in_specs=[
            pl.BlockSpec((BR, D), lambda i: (i, 0)),
            pl.BlockSpec((BR, D), lambda i: (i, 0)),
            pl.BlockSpec((2, BR, 1), lambda i: (0, i, 0)),
            pl.BlockSpec((2, D, D), lambda i: (0, 0, 0)),
        ],
        out_specs=[
            pl.BlockSpec((2, BR, D), lambda i: (0, i, 0)),
            pl.BlockSpec((BR, 1), lambda i: (i, 0)),
        ],
        out_shape=[
            jax.ShapeDtypeStruct((2, NP, D), jnp.float32),
            jax.ShapeDtypeStruct((NP, 1), jnp.float32),
        ],
    )(z_pad, x_pad, deg2, w1)


def _tc2_body(agg_ref, g_ref, dinv_ref, b_ref, w_ref, out_ref):
    dinv = dinv_ref[...]
    for c in range(2):
        u = dinv * (agg_ref[c] + g_ref[c]) + b_ref[c]
        u = jnp.where(u >= 0, u, u * SLOPE)
        out_ref[c] = dinv * jnp.dot(u, w_ref[c],
                                    preferred_element_type=jnp.float32)


def _tc2(agg1, g1, dinv, b1, w2):
    return pl.pallas_call(
        _tc2_body,
        grid=(GRID,),
        in_specs=[
            pl.BlockSpec((2, BR, D), lambda i: (0, i, 0)),
            pl.BlockSpec((2, BR, D), lambda i: (0, i, 0)),
            pl.BlockSpec((BR, 1), lambda i: (i, 0)),
            pl.BlockSpec((2, D), lambda i: (0, 0)),
            pl.BlockSpec((2, D, D), lambda i: (0, 0, 0)),
        ],
        out_specs=pl.BlockSpec((2, BR, D), lambda i: (0, i, 0)),
        out_shape=jax.ShapeDtypeStruct((2, NP, D), jnp.float32),
    )(agg1, g1, dinv, b1, w2)


def _tc3_body(agg_ref, g_ref, dinv_ref, b_ref, wo_ref, bo_ref, out_ref):
    dinv = dinv_ref[...]
    zz = jnp.tanh(dinv * (agg_ref[0] + g_ref[0]) + b_ref[0])
    xx = jnp.tanh(dinv * (agg_ref[1] + g_ref[1]) + b_ref[1])
    out_ref[...] = jnp.dot(zz * xx, wo_ref[...],
                           preferred_element_type=jnp.float32) + bo_ref[...]


def _tc3(agg2, g2, dinv, b2, Wo, bo):
    return pl.pallas_call(
        _tc3_body,
        grid=(GRID,),
        in_specs=[
            pl.BlockSpec((2, BR, D), lambda i: (0, i, 0)),
            pl.BlockSpec((2, BR, D), lambda i: (0, i, 0)),
            pl.BlockSpec((BR, 1), lambda i: (i, 0)),
            pl.BlockSpec((2, D), lambda i: (0, 0)),
            pl.BlockSpec((D, 1), lambda i: (0, 0)),
            pl.BlockSpec((1,), lambda i: (0,)),
        ],
        out_specs=pl.BlockSpec((BR, 1), lambda i: (i, 0)),
        out_shape=jax.ShapeDtypeStruct((NP, 1), jnp.float32),
    )(agg2, g2, dinv, b2, Wo, bo)


@jax.jit
def kernel(z, x, edge_index, We1, be1, We2, be2, Wf1, bf1, Wf2, bf2, Wo, bo):
    src_p = jnp.concatenate(
        [edge_index[0], jnp.zeros((EP - E,), jnp.int32)]).reshape(ECH, CH)
    dst_p = jnp.concatenate(
        [edge_index[1], jnp.full((EP - E,), N, jnp.int32)]).reshape(ECH, CH)

    z_pad = jnp.pad(z, ((0, NP - N), (0, 0)))
    x_pad = jnp.pad(x, ((0, NP - N), (0, 0)))
    w1 = jnp.stack([We1, Wf1])
    w2 = jnp.stack([We2, Wf2])
    b1 = jnp.stack([be1, bf1])
    b2 = jnp.stack([be2, bf2])

    deg2 = _sc_deg(dst_p.reshape(EP))
    deg2 = deg2[:, :, None]

    g1, dinv = _tc1(z_pad, x_pad, deg2, w1)
    agg1 = _sc_agg(g1, src_p, dst_p)
    g2 = _tc2(agg1, g1, dinv, b1, w2)
    agg2 = _sc_agg(g2, src_p, dst_p)
    out = _tc3(agg2, g2, dinv, b2, Wo, bo)
    return out[:N]


# speedup vs baseline: 1.0456x; 1.0425x over previous
import functools

import jax
import jax.numpy as jnp
from jax import lax
from jax.experimental import pallas as pl
from jax.experimental.pallas import tpu as pltpu
from jax.experimental.pallas import tpu_sc as plsc

N = 10000
NP = 10240
E = 320000
D = 128
CH = 128
ECH = 2560
EP = ECH * CH
BR = 512
GRID = NP // BR
RPT = NP // 16
SLOPE = (1.0 / 8.0 + 1.0 / 3.0) / 2.0

NJ = ECH // 16
BT = 16
NB = NJ // BT
NRING = 2
NJD = ECH // 32


def _mesh():
    return plsc.VectorSubcoreMesh(core_axis_name="c", subcore_axis_name="s")


def _sc_deg_body(dst_hbm, out_hbm, didx, ones_v, zbuf, acc):
    cid = lax.axis_index("c")
    sid = lax.axis_index("s")
    wid = cid * 16 + sid

    for l in range(8):
        ones_v[pl.ds(l * 16, 16)] = jnp.full((16,), 1.0, jnp.float32)
    zeros16 = jnp.zeros((16,), jnp.float32)

    @pl.loop(0, RPT // 16)
    def _zero(i):
        zbuf[pl.ds(i * 16, 16)] = zeros16

    pltpu.sync_copy(zbuf, acc.at[pl.ds(sid * RPT, RPT)])
    plsc.subcore_barrier()

    @pl.loop(0, NJD)
    def _edges(j):
        off = (wid + 32 * j) * CH
        pltpu.sync_copy(dst_hbm.at[pl.ds(off, CH)], didx)
        pltpu.sync_copy(ones_v, acc.at[didx], add=True)

    plsc.subcore_barrier()
    pltpu.sync_copy(acc.at[pl.ds(sid * RPT, RPT)],
                    out_hbm.at[cid, pl.ds(sid * RPT, RPT)])


def _sc_deg(dst):
    f = functools.partial(
        pl.kernel,
        out_type=jax.ShapeDtypeStruct((2, NP), jnp.float32),
        mesh=_mesh(),
        scratch_types=[
            pltpu.VMEM((CH,), jnp.int32),
            pltpu.VMEM((CH,), jnp.float32),
            pltpu.VMEM((RPT,), jnp.float32),
            pltpu.VMEM_SHARED((NP,), jnp.float32),
        ],
    )(_sc_deg_body)
    return f(dst)


def _sc_agg_body(g_hbm, src_hbm, dst_hbm, out_hbm,
                 sidxA, didxA, sidxB, didxB,
                 rows0, rows1,
                 acc, isem, gsem0, gsem1):
    cid = lax.axis_index("c")
    sid = lax.axis_index("s")
    gv = g_hbm.at[cid]

    sbuf = (sidxA, sidxB)
    dbuf = (didxA, didxB)
    rbuf = (rows0, rows1)
    gsem = (gsem0, gsem1)

    zeros16 = jnp.zeros((16,), jnp.float32)

    @pl.loop(0, CH)
    def _zrow(r):
        for l in range(D // 16):
            rows0[r, pl.ds(l * 16, 16)] = zeros16

    for k in range(RPT // CH):
        pltpu.sync_copy(rows0, acc.at[pl.ds(sid * RPT + k * CH, CH)])
    plsc.subcore_barrier()

    cbase = sid * NJ

    def load_idx_batch(t, ib):
        row = cbase + t * BT
        pltpu.async_copy(src_hbm.at[pl.ds(row, BT)], sbuf[ib], isem)
        pltpu.async_copy(dst_hbm.at[pl.ds(row, BT)], dbuf[ib], isem)

    def wait_idx_batch(ib):
        pltpu.make_async_copy(src_hbm.at[pl.ds(0, BT)], sbuf[ib], isem).wait()
        pltpu.make_async_copy(dst_hbm.at[pl.ds(0, BT)], dbuf[ib], isem).wait()

    def start_gather(ib, k, b):
        pltpu.async_copy(gv.at[sbuf[ib].at[k]], rbuf[b], gsem[b])

    def wait_gather(ib, k, b):
        pltpu.make_async_copy(gv.at[sbuf[ib].at[k]], rbuf[b], gsem[b]).wait()

    def scatter(ib, k, b):
        pltpu.sync_copy(rbuf[b], acc.at[dbuf[ib].at[k]], add=True)

    load_idx_batch(0, 0)
    wait_idx_batch(0)
    start_gather(0, 0, 0)
    for k in range(BT):
        b = k % NRING
        nb_ = (k + 1) % NRING
        if k == 2:
            load_idx_batch(1, 1)
        if k == BT - 2:
            wait_idx_batch(1)
        if k < BT - 1:
            start_gather(0, k + 1, nb_)
        else:
            start_gather(1, 0, nb_)
        wait_gather(0, k, b)
        scatter(0, k, b)

    @pl.loop(1, NB)
    def _batch(t):
        tb = t % 2
        for parity in range(2):
            @pl.when(tb == parity)
            def _run(parity=parity):
                ib = parity
                nib = 1 - parity
                for k in range(BT):
                    b = k % NRING
                    nb_ = (k + 1) % NRING
                    if k == 2:
                        @pl.when(t + 1 < NB)
                        def _pf():
                            load_idx_batch(t + 1, nib)
                    if k == BT - 2:
                        @pl.when(t + 1 < NB)
                        def _wf():
                            wait_idx_batch(nib)
                    if k < BT - 1:
                        start_gather(ib, k + 1, nb_)
                    else:
                        @pl.when(t + 1 < NB)
                        def _ng():
                            start_gather(nib, 0, nb_)
                    wait_gather(ib, k, b)
                    scatter(ib, k, b)

    plsc.subcore_barrier()
    for k in range(RPT // CH):
        pltpu.sync_copy(acc.at[pl.ds(sid * RPT + k * CH, CH)],
                        out_hbm.at[cid].at[pl.ds(sid * RPT + k * CH, CH)])


def _sc_agg(g, src2d, dst2d):
    f = functools.partial(
        pl.kernel,
        out_type=jax.ShapeDtypeStruct((2, NP, D), jnp.float32),
        mesh=_mesh(),
        scratch_types=[
            pltpu.VMEM((BT, CH), jnp.int32),
            pltpu.VMEM((BT, CH), jnp.int32),
            pltpu.VMEM((BT, CH), jnp.int32),
            pltpu.VMEM((BT, CH), jnp.int32),
            pltpu.VMEM((CH, D), jnp.float32),
            pltpu.VMEM((CH, D), jnp.float32),
            pltpu.VMEM_SHARED((NP, D), jnp.float32),
            pltpu.SemaphoreType.DMA,
            pltpu.SemaphoreType.DMA,
            pltpu.SemaphoreType.DMA,
        ],
    )(_sc_agg_body)
    return f(g, src2d, dst2d)


def _tc1_body(z_ref, x_ref, d2_ref, w_ref, g_ref, dinv_ref):
    deg = d2_ref[0] + d2_ref[1] + 1.0
    dinv = lax.rsqrt(deg)
    dinv_ref[...] = dinv
    g_ref[0] = dinv * jnp.dot(z_ref[...], w_ref[0],
                              preferred_element_type=jnp.float32)
    g_ref[1] = dinv * jnp.dot(x_ref[...], w_ref[1],
                              preferred_element_type=jnp.float32)


def _tc1(z_pad, x_pad, deg2, w1):
    return pl.pallas_call(
        _tc1_body,
        grid=(GRID,),
        in_specs=[
            pl.BlockSpec((BR, D), lambda i: (i, 0)),
            pl.BlockSpec((BR, D), lambda i: (i, 0)),
            pl.BlockSpec((2, BR, 1), lambda i: (0, i, 0)),
            pl.BlockSpec((2, D, D), lambda i: (0, 0, 0)),
        ],
        out_specs=[
            pl.BlockSpec((2, BR, D), lambda i: (0, i, 0)),
            pl.BlockSpec((BR, 1), lambda i: (i, 0)),
        ],
        out_shape=[
            jax.ShapeDtypeStruct((2, NP, D), jnp.float32),
            jax.ShapeDtypeStruct((NP, 1), jnp.float32),
        ],
    )(z_pad, x_pad, deg2, w1)


def _tc2_body(agg_ref, g_ref, dinv_ref, b_ref, w_ref, out_ref):
    dinv = dinv_ref[...]
    for c in range(2):
        u = dinv * (agg_ref[c] + g_ref[c]) + b_ref[c]
        u = jnp.where(u >= 0, u, u * SLOPE)
        out_ref[c] = dinv * jnp.dot(u, w_ref[c],
                                    preferred_element_type=jnp.float32)


def _tc2(agg1, g1, dinv, b1, w2):
    return pl.pallas_call(
        _tc2_body,
        grid=(GRID,),
        in_specs=[
            pl.BlockSpec((2, BR, D), lambda i: (0, i, 0)),
            pl.BlockSpec((2, BR, D), lambda i: (0, i, 0)),
            pl.BlockSpec((BR, 1), lambda i: (i, 0)),
            pl.BlockSpec((2, D), lambda i: (0, 0)),
            pl.BlockSpec((2, D, D), lambda i: (0, 0, 0)),
        ],
        out_specs=pl.BlockSpec((2, BR, D), lambda i: (0, i, 0)),
        out_shape=jax.ShapeDtypeStruct((2, NP, D), jnp.float32),
    )(agg1, g1, dinv, b1, w2)


def _tc3_body(agg_ref, g_ref, dinv_ref, b_ref, wo_ref, bo_ref, out_ref):
    dinv = dinv_ref[...]
    zz = jnp.tanh(dinv * (agg_ref[0] + g_ref[0]) + b_ref[0])
    xx = jnp.tanh(dinv * (agg_ref[1] + g_ref[1]) + b_ref[1])
    out_ref[...] = jnp.dot(zz * xx, wo_ref[...],
                           preferred_element_type=jnp.float32) + bo_ref[...]


def _tc3(agg2, g2, dinv, b2, Wo, bo):
    return pl.pallas_call(
        _tc3_body,
        grid=(GRID,),
        in_specs=[
            pl.BlockSpec((2, BR, D), lambda i: (0, i, 0)),
            pl.BlockSpec((2, BR, D), lambda i: (0, i, 0)),
            pl.BlockSpec((BR, 1), lambda i: (i, 0)),
            pl.BlockSpec((2, D), lambda i: (0, 0)),
            pl.BlockSpec((D, 1), lambda i: (0, 0)),
            pl.BlockSpec((1,), lambda i: (0,)),
        ],
        out_specs=pl.BlockSpec((BR, 1), lambda i: (i, 0)),
        out_shape=jax.ShapeDtypeStruct((NP, 1), jnp.float32),
    )(agg2, g2, dinv, b2, Wo, bo)


@jax.jit
def kernel(z, x, edge_index, We1, be1, We2, be2, Wf1, bf1, Wf2, bf2, Wo, bo):
    src_p = jnp.concatenate(
        [edge_index[0], jnp.zeros((EP - E,), jnp.int32)]).reshape(ECH, CH)
    pad_dst = N + jnp.arange(EP - E, dtype=jnp.int32) % (NP - N)
    dst_p = jnp.concatenate([edge_index[1], pad_dst]).reshape(ECH, CH)

    z_pad = jnp.pad(z, ((0, NP - N), (0, 0)))
    x_pad = jnp.pad(x, ((0, NP - N), (0, 0)))
    w1 = jnp.stack([We1, Wf1])
    w2 = jnp.stack([We2, Wf2])
    b1 = jnp.stack([be1, bf1])
    b2 = jnp.stack([be2, bf2])

    deg2 = _sc_deg(dst_p.reshape(EP))
    deg2 = deg2[:, :, None]

    g1, dinv = _tc1(z_pad, x_pad, deg2, w1)
    agg1 = _sc_agg(g1, src_p, dst_p)
    g2 = _tc2(agg1, g1, dinv, b1, w2)
    agg2 = _sc_agg(g2, src_p, dst_p)
    out = _tc3(agg2, g2, dinv, b2, Wo, bo)
    return out[:N]


# speedup vs baseline: 1.1180x; 1.0693x over previous
import functools

import jax
import jax.numpy as jnp
from jax import lax
from jax.experimental import pallas as pl
from jax.experimental.pallas import tpu as pltpu
from jax.experimental.pallas import tpu_sc as plsc

N = 10000
NP = 10240
E = 320000
D = 128
CH = 128
ECH = 2560
EP = ECH * CH
BR = 512
GRID = NP // BR
RPT = NP // 16
SLOPE = (1.0 / 8.0 + 1.0 / 3.0) / 2.0

NJ = ECH // 16
BT = 16
NB = NJ // BT
NRING = 2
NJD = ECH // 32


def _mesh():
    return plsc.VectorSubcoreMesh(core_axis_name="c", subcore_axis_name="s")


def _sc_deg_body(dst_hbm, out_hbm, didx, ones_v, zbuf, acc):
    cid = lax.axis_index("c")
    sid = lax.axis_index("s")
    wid = cid * 16 + sid

    for l in range(8):
        ones_v[pl.ds(l * 16, 16)] = jnp.full((16,), 1.0, jnp.float32)
    zeros16 = jnp.zeros((16,), jnp.float32)

    @pl.loop(0, RPT // 16)
    def _zero(i):
        zbuf[pl.ds(i * 16, 16)] = zeros16

    pltpu.sync_copy(zbuf, acc.at[pl.ds(sid * RPT, RPT)])
    plsc.subcore_barrier()

    @pl.loop(0, NJD)
    def _edges(j):
        off = (wid + 32 * j) * CH
        pltpu.sync_copy(dst_hbm.at[pl.ds(off, CH)], didx)
        pltpu.sync_copy(ones_v, acc.at[didx], add=True)

    plsc.subcore_barrier()
    pltpu.sync_copy(acc.at[pl.ds(sid * RPT, RPT)],
                    out_hbm.at[cid, pl.ds(sid * RPT, RPT)])


def _sc_deg(dst):
    f = functools.partial(
        pl.kernel,
        out_type=jax.ShapeDtypeStruct((2, NP), jnp.float32),
        mesh=_mesh(),
        scratch_types=[
            pltpu.VMEM((CH,), jnp.int32),
            pltpu.VMEM((CH,), jnp.float32),
            pltpu.VMEM((RPT,), jnp.float32),
            pltpu.VMEM_SHARED((NP,), jnp.float32),
        ],
    )(_sc_deg_body)
    return f(dst)


def _sc_agg_body(g_hbm, src_hbm, dst_hbm, out_hbm,
                 sidxA, didxA, sidxB, didxB,
                 rows0, rows1,
                 acc, isem, gsem0, gsem1):
    cid = lax.axis_index("c")
    sid = lax.axis_index("s")
    gv = g_hbm.at[cid]

    sbuf = (sidxA, sidxB)
    dbuf = (didxA, didxB)
    rbuf = (rows0, rows1)
    gsem = (gsem0, gsem1)

    zeros16 = jnp.zeros((16,), jnp.float32)

    @pl.loop(0, CH)
    def _zrow(r):
        for l in range(D // 16):
            rows0[r, pl.ds(l * 16, 16)] = zeros16

    for k in range(RPT // CH):
        pltpu.sync_copy(rows0, acc.at[pl.ds(sid * RPT + k * CH, CH)])
    plsc.subcore_barrier()

    def load_idx(j, b):
        off = (sid + 16 * j) * CH
        pltpu.sync_copy(src_hbm.at[pl.ds(off, CH)], sbuf[b])
        pltpu.sync_copy(dst_hbm.at[pl.ds(off, CH)], dbuf[b])

    def start_gather(b):
        pltpu.async_copy(gv.at[sbuf[b]], rbuf[b], gsem[b])

    def finish(b):
        pltpu.make_async_copy(gv.at[sbuf[b]], rbuf[b], gsem[b]).wait()
        pltpu.sync_copy(rbuf[b], acc.at[dbuf[b]], add=True)

    load_idx(0, 0)
    start_gather(0)

    @pl.loop(0, NJ, step=2)
    def _edges(j):
        load_idx(j + 1, 1)
        start_gather(1)
        finish(0)

        @pl.when(j + 2 < NJ)
        def _():
            load_idx(j + 2, 0)
            start_gather(0)

        finish(1)

    plsc.subcore_barrier()
    for k in range(RPT // CH):
        pltpu.sync_copy(acc.at[pl.ds(sid * RPT + k * CH, CH)],
                        out_hbm.at[cid].at[pl.ds(sid * RPT + k * CH, CH)])


def _sc_agg(g, src2d, dst2d):
    f = functools.partial(
        pl.kernel,
        out_type=jax.ShapeDtypeStruct((2, NP, D), jnp.float32),
        mesh=_mesh(),
        scratch_types=[
            pltpu.VMEM((CH,), jnp.int32),
            pltpu.VMEM((CH,), jnp.int32),
            pltpu.VMEM((CH,), jnp.int32),
            pltpu.VMEM((CH,), jnp.int32),
            pltpu.VMEM((CH, D), jnp.float32),
            pltpu.VMEM((CH, D), jnp.float32),
            pltpu.VMEM_SHARED((NP, D), jnp.float32),
            pltpu.SemaphoreType.DMA,
            pltpu.SemaphoreType.DMA,
            pltpu.SemaphoreType.DMA,
        ],
    )(_sc_agg_body)
    return f(g, src2d, dst2d)


def _tc1_body(z_ref, x_ref, d2_ref, w_ref, g_ref, dinv_ref):
    deg = d2_ref[0] + d2_ref[1] + 1.0
    dinv = lax.rsqrt(deg)
    dinv_ref[...] = dinv
    g_ref[0] = dinv * jnp.dot(z_ref[...], w_ref[0],
                              preferred_element_type=jnp.float32)
    g_ref[1] = dinv * jnp.dot(x_ref[...], w_ref[1],
                              preferred_element_type=jnp.float32)


def _tc1(z_pad, x_pad, deg2, w1):
    return pl.pallas_call(
        _tc1_body,
        grid=(GRID,),
        in_specs=[
            pl.BlockSpec((BR, D), lambda i: (i, 0)),
            pl.BlockSpec((BR, D), lambda i: (i, 0)),
            pl.BlockSpec((2, BR, 1), lambda i: (0, i, 0)),
            pl.BlockSpec((2, D, D), lambda i: (0, 0, 0)),
        ],
        out_specs=[
            pl.BlockSpec((2, BR, D), lambda i: (0, i, 0)),
            pl.BlockSpec((BR, 1), lambda i: (i, 0)),
        ],
        out_shape=[
            jax.ShapeDtypeStruct((2, NP, D), jnp.float32),
            jax.ShapeDtypeStruct((NP, 1), jnp.float32),
        ],
    )(z_pad, x_pad, deg2, w1)


def _tc2_body(agg_ref, g_ref, dinv_ref, b_ref, w_ref, out_ref):
    dinv = dinv_ref[...]
    for c in range(2):
        u = dinv * (agg_ref[c] + g_ref[c]) + b_ref[c]
        u = jnp.where(u >= 0, u, u * SLOPE)
        out_ref[c] = dinv * jnp.dot(u, w_ref[c],
                                    preferred_element_type=jnp.float32)


def _tc2(agg1, g1, dinv, b1, w2):
    return pl.pallas_call(
        _tc2_body,
        grid=(GRID,),
        in_specs=[
            pl.BlockSpec((2, BR, D), lambda i: (0, i, 0)),
            pl.BlockSpec((2, BR, D), lambda i: (0, i, 0)),
            pl.BlockSpec((BR, 1), lambda i: (i, 0)),
            pl.BlockSpec((2, D), lambda i: (0, 0)),
            pl.BlockSpec((2, D, D), lambda i: (0, 0, 0)),
        ],
        out_specs=pl.BlockSpec((2, BR, D), lambda i: (0, i, 0)),
        out_shape=jax.ShapeDtypeStruct((2, NP, D), jnp.float32),
    )(agg1, g1, dinv, b1, w2)


def _tc3_body(agg_ref, g_ref, dinv_ref, b_ref, wo_ref, bo_ref, out_ref):
    dinv = dinv_ref[...]
    zz = jnp.tanh(dinv * (agg_ref[0] + g_ref[0]) + b_ref[0])
    xx = jnp.tanh(dinv * (agg_ref[1] + g_ref[1]) + b_ref[1])
    out_ref[...] = jnp.dot(zz * xx, wo_ref[...],
                           preferred_element_type=jnp.float32) + bo_ref[...]


def _tc3(agg2, g2, dinv, b2, Wo, bo):
    return pl.pallas_call(
        _tc3_body,
        grid=(GRID,),
        in_specs=[
            pl.BlockSpec((2, BR, D), lambda i: (0, i, 0)),
            pl.BlockSpec((2, BR, D), lambda i: (0, i, 0)),
            pl.BlockSpec((BR, 1), lambda i: (i, 0)),
            pl.BlockSpec((2, D), lambda i: (0, 0)),
            pl.BlockSpec((D, 1), lambda i: (0, 0)),
            pl.BlockSpec((1,), lambda i: (0,)),
        ],
        out_specs=pl.BlockSpec((BR, 1), lambda i: (i, 0)),
        out_shape=jax.ShapeDtypeStruct((NP, 1), jnp.float32),
    )(agg2, g2, dinv, b2, Wo, bo)


@jax.jit
def kernel(z, x, edge_index, We1, be1, We2, be2, Wf1, bf1, Wf2, bf2, Wo, bo):
    src_p = jnp.concatenate(
        [edge_index[0], jnp.zeros((EP - E,), jnp.int32)]).reshape(ECH, CH)
    pad_dst = N + jnp.arange(EP - E, dtype=jnp.int32) % (NP - N)
    dst_p = jnp.concatenate([edge_index[1], pad_dst]).reshape(ECH, CH)

    z_pad = jnp.pad(z, ((0, NP - N), (0, 0)))
    x_pad = jnp.pad(x, ((0, NP - N), (0, 0)))
    w1 = jnp.stack([We1, Wf1])
    w2 = jnp.stack([We2, Wf2])
    b1 = jnp.stack([be1, bf1])
    b2 = jnp.stack([be2, bf2])

    deg2 = _sc_deg(dst_p.reshape(EP))
    deg2 = deg2[:, :, None]

    g1, dinv = _tc1(z_pad, x_pad, deg2, w1)
    agg1 = _sc_agg(g1, src_p.reshape(EP), dst_p.reshape(EP))
    g2 = _tc2(agg1, g1, dinv, b1, w2)
    agg2 = _sc_agg(g2, src_p.reshape(EP), dst_p.reshape(EP))
    out = _tc3(agg2, g2, dinv, b2, Wo, bo)
    return out[:N]


# speedup vs baseline: 1.9929x; 1.7825x over previous
import functools

import jax
import jax.numpy as jnp
from jax import lax
from jax.experimental import pallas as pl
from jax.experimental.pallas import tpu as pltpu
from jax.experimental.pallas import tpu_sc as plsc

N = 10000
NP = 10240
E = 320000
D = 128
CH = 128
ECH = 2560
EP = ECH * CH
BR = 512
GRID = NP // BR
RPT = NP // 16
SLOPE = (1.0 / 8.0 + 1.0 / 3.0) / 2.0

NJ = ECH // 16
BT = 16
NB = NJ // BT
NRING = 2
NJD = ECH // 32


def _mesh():
    return plsc.VectorSubcoreMesh(core_axis_name="c", subcore_axis_name="s")


def _sc_deg_body(dst_hbm, out_hbm, didx, ones_v, zbuf, acc):
    cid = lax.axis_index("c")
    sid = lax.axis_index("s")
    wid = cid * 16 + sid

    for l in range(8):
        ones_v[pl.ds(l * 16, 16)] = jnp.full((16,), 1.0, jnp.float32)
    zeros16 = jnp.zeros((16,), jnp.float32)

    @pl.loop(0, RPT // 16)
    def _zero(i):
        zbuf[pl.ds(i * 16, 16)] = zeros16

    pltpu.sync_copy(zbuf, acc.at[pl.ds(sid * RPT, RPT)])
    plsc.subcore_barrier()

    @pl.loop(0, NJD)
    def _edges(j):
        off = (wid + 32 * j) * CH
        pltpu.sync_copy(dst_hbm.at[pl.ds(off, CH)], didx)
        pltpu.sync_copy(ones_v, acc.at[didx], add=True)

    plsc.subcore_barrier()
    pltpu.sync_copy(acc.at[pl.ds(sid * RPT, RPT)],
                    out_hbm.at[cid, pl.ds(sid * RPT, RPT)])


def _sc_deg(dst):
    f = functools.partial(
        pl.kernel,
        out_type=jax.ShapeDtypeStruct((2, NP), jnp.float32),
        mesh=_mesh(),
        scratch_types=[
            pltpu.VMEM((CH,), jnp.int32),
            pltpu.VMEM((CH,), jnp.float32),
            pltpu.VMEM((RPT,), jnp.float32),
            pltpu.VMEM_SHARED((NP,), jnp.float32),
        ],
    )(_sc_deg_body)
    return f(dst)


def _sc_agg_body(g_hbm, src_hbm, dst_hbm, out_hbm,
                 sidxA, didxA, sidxB, didxB,
                 rows0, rows1,
                 acc, isem, gsem0, gsem1):
    cid = lax.axis_index("c")
    sid = lax.axis_index("s")
    gv = g_hbm.at[cid]

    sbuf = (sidxA, sidxB)
    dbuf = (didxA, didxB)
    rbuf = (rows0, rows1)
    gsem = (gsem0, gsem1)

    zeros16 = jnp.zeros((16,), jnp.float32)

    @pl.loop(0, CH)
    def _zrow(r):
        for l in range(D // 16):
            rows0[r, pl.ds(l * 16, 16)] = zeros16

    for k in range(RPT // CH):
        pltpu.sync_copy(rows0, acc.at[pl.ds(sid * RPT + k * CH, CH)])
    plsc.subcore_barrier()

    def load_idx(j, b):
        off = (sid + 16 * j) * CH
        pltpu.sync_copy(src_hbm.at[pl.ds(off, CH)], sbuf[b])
        pltpu.sync_copy(dst_hbm.at[pl.ds(off, CH)], dbuf[b])

    def start_gather(b):
        pltpu.async_copy(gv.at[sbuf[b]], rbuf[b], gsem[b])

    def finish(b):
        pltpu.make_async_copy(gv.at[sbuf[b]], rbuf[b], gsem[b]).wait()
        pltpu.sync_copy(rbuf[b], acc.at[dbuf[b]], add=True)

    load_idx(0, 0)
    start_gather(0)

    @pl.loop(0, NJ, step=2)
    def _edges(j):
        load_idx(j + 1, 1)
        start_gather(1)
        finish(0)

        @pl.when(j + 2 < NJ)
        def _():
            load_idx(j + 2, 0)
            start_gather(0)

        finish(1)

    plsc.subcore_barrier()
    for k in range(RPT // CH):
        pltpu.sync_copy(acc.at[pl.ds(sid * RPT + k * CH, CH)],
                        out_hbm.at[cid].at[pl.ds(sid * RPT + k * CH, CH)])


def _sc_agg(g, src2d, dst2d):
    f = functools.partial(
        pl.kernel,
        out_type=jax.ShapeDtypeStruct((2, NP, D), jnp.float32),
        mesh=_mesh(),
        scratch_types=[
            pltpu.VMEM((CH,), jnp.int32),
            pltpu.VMEM((CH,), jnp.int32),
            pltpu.VMEM((CH,), jnp.int32),
            pltpu.VMEM((CH,), jnp.int32),
            pltpu.VMEM((CH, D), jnp.float32),
            pltpu.VMEM((CH, D), jnp.float32),
            pltpu.VMEM_SHARED((NP, D), jnp.float32),
            pltpu.SemaphoreType.DMA,
            pltpu.SemaphoreType.DMA,
            pltpu.SemaphoreType.DMA,
        ],
    )(_sc_agg_body)
    return f(g, src2d, dst2d)


def _tc1_body(z_ref, x_ref, d2_ref, w_ref, g_ref, dinv_ref):
    deg = d2_ref[0] + d2_ref[1] + 1.0
    dinv = lax.rsqrt(deg)
    dinv_ref[...] = dinv
    g_ref[0] = dinv * jnp.dot(z_ref[...], w_ref[0],
                              preferred_element_type=jnp.float32)
    g_ref[1] = dinv * jnp.dot(x_ref[...], w_ref[1],
                              preferred_element_type=jnp.float32)


def _tc1(z_pad, x_pad, deg2, w1):
    return pl.pallas_call(
        _tc1_body,
        grid=(GRID,),
        in_specs=[
            pl.BlockSpec((BR, D), lambda i: (i, 0)),
            pl.BlockSpec((BR, D), lambda i: (i, 0)),
            pl.BlockSpec((2, BR, 1), lambda i: (0, i, 0)),
            pl.BlockSpec((2, D, D), lambda i: (0, 0, 0)),
        ],
        out_specs=[
            pl.BlockSpec((2, BR, D), lambda i: (0, i, 0)),
            pl.BlockSpec((BR, 1), lambda i: (i, 0)),
        ],
        out_shape=[
            jax.ShapeDtypeStruct((2, NP, D), jnp.float32),
            jax.ShapeDtypeStruct((NP, 1), jnp.float32),
        ],
    )(z_pad, x_pad, deg2, w1)


def _tc2_body(agg_ref, g_ref, dinv_ref, b_ref, w_ref, out_ref):
    dinv = dinv_ref[...]
    for c in range(2):
        u = dinv * (agg_ref[c] + g_ref[c]) + b_ref[c]
        u = jnp.where(u >= 0, u, u * SLOPE)
        out_ref[c] = dinv * jnp.dot(u, w_ref[c],
                                    preferred_element_type=jnp.float32)


def _tc2(agg1, g1, dinv, b1, w2):
    return pl.pallas_call(
        _tc2_body,
        grid=(GRID,),
        in_specs=[
            pl.BlockSpec((2, BR, D), lambda i: (0, i, 0)),
            pl.BlockSpec((2, BR, D), lambda i: (0, i, 0)),
            pl.BlockSpec((BR, 1), lambda i: (i, 0)),
            pl.BlockSpec((2, D), lambda i: (0, 0)),
            pl.BlockSpec((2, D, D), lambda i: (0, 0, 0)),
        ],
        out_specs=pl.BlockSpec((2, BR, D), lambda i: (0, i, 0)),
        out_shape=jax.ShapeDtypeStruct((2, NP, D), jnp.float32),
    )(agg1, g1, dinv, b1, w2)


def _tc3_body(agg_ref, g_ref, dinv_ref, b_ref, wo_ref, bo_ref, out_ref):
    dinv = dinv_ref[...]
    zz = jnp.tanh(dinv * (agg_ref[0] + g_ref[0]) + b_ref[0])
    xx = jnp.tanh(dinv * (agg_ref[1] + g_ref[1]) + b_ref[1])
    out_ref[...] = jnp.dot(zz * xx, wo_ref[...],
                           preferred_element_type=jnp.float32) + bo_ref[...]


def _tc3(agg2, g2, dinv, b2, Wo, bo):
    return pl.pallas_call(
        _tc3_body,
        grid=(GRID,),
        in_specs=[
            pl.BlockSpec((2, BR, D), lambda i: (0, i, 0)),
            pl.BlockSpec((2, BR, D), lambda i: (0, i, 0)),
            pl.BlockSpec((BR, 1), lambda i: (i, 0)),
            pl.BlockSpec((2, D), lambda i: (0, 0)),
            pl.BlockSpec((D, 1), lambda i: (0, 0)),
            pl.BlockSpec((1,), lambda i: (0,)),
        ],
        out_specs=pl.BlockSpec((BR, 1), lambda i: (i, 0)),
        out_shape=jax.ShapeDtypeStruct((NP, 1), jnp.float32),
    )(agg2, g2, dinv, b2, Wo, bo)


@jax.jit
def kernel(z, x, edge_index, We1, be1, We2, be2, Wf1, bf1, Wf2, bf2, Wo, bo):
    pad_idx = N + jnp.arange(EP - E, dtype=jnp.int32) % (NP - N)
    src_p = jnp.concatenate([edge_index[0], pad_idx]).reshape(ECH, CH)
    dst_p = jnp.concatenate([edge_index[1], pad_idx]).reshape(ECH, CH)

    z_pad = jnp.pad(z, ((0, NP - N), (0, 0)))
    x_pad = jnp.pad(x, ((0, NP - N), (0, 0)))
    w1 = jnp.stack([We1, Wf1])
    w2 = jnp.stack([We2, Wf2])
    b1 = jnp.stack([be1, bf1])
    b2 = jnp.stack([be2, bf2])

    deg2 = _sc_deg(dst_p.reshape(EP))
    deg2 = deg2[:, :, None]

    g1, dinv = _tc1(z_pad, x_pad, deg2, w1)
    agg1 = _sc_agg(g1, src_p.reshape(EP), dst_p.reshape(EP))
    g2 = _tc2(agg1, g1, dinv, b1, w2)
    agg2 = _sc_agg(g2, src_p.reshape(EP), dst_p.reshape(EP))
    out = _tc3(agg2, g2, dinv, b2, Wo, bo)
    return out[:N]


# speedup vs baseline: 2.6062x; 1.3078x over previous
import functools

import jax
import jax.numpy as jnp
from jax import lax
from jax.experimental import pallas as pl
from jax.experimental.pallas import tpu as pltpu
from jax.experimental.pallas import tpu_sc as plsc

N = 10000
NP = 10240
E = 320000
D = 128
CH = 128
ECH = 2560
EP = ECH * CH
BR = 512
GRID = NP // BR
RPT = NP // 16
SLOPE = (1.0 / 8.0 + 1.0 / 3.0) / 2.0

NJ = ECH // 16
BT = 16
NB = NJ // BT
NRING = 2
NJD = ECH // 32


def _mesh():
    return plsc.VectorSubcoreMesh(core_axis_name="c", subcore_axis_name="s")


def _sc_deg_body(dst_hbm, out_hbm, didx, ones_v, zbuf, acc):
    cid = lax.axis_index("c")
    sid = lax.axis_index("s")
    wid = cid * 16 + sid

    for l in range(8):
        ones_v[pl.ds(l * 16, 16)] = jnp.full((16,), 1.0, jnp.float32)
    zeros16 = jnp.zeros((16,), jnp.float32)

    @pl.loop(0, RPT // 16)
    def _zero(i):
        zbuf[pl.ds(i * 16, 16)] = zeros16

    pltpu.sync_copy(zbuf, acc.at[pl.ds(sid * RPT, RPT)])
    plsc.subcore_barrier()

    @pl.loop(0, NJD)
    def _edges(j):
        off = (wid + 32 * j) * CH
        pltpu.sync_copy(dst_hbm.at[pl.ds(off, CH)], didx)
        pltpu.sync_copy(ones_v, acc.at[didx], add=True)

    plsc.subcore_barrier()
    pltpu.sync_copy(acc.at[pl.ds(sid * RPT, RPT)],
                    out_hbm.at[cid, pl.ds(sid * RPT, RPT)])


def _sc_deg(dst):
    f = functools.partial(
        pl.kernel,
        out_type=jax.ShapeDtypeStruct((2, NP), jnp.float32),
        mesh=_mesh(),
        scratch_types=[
            pltpu.VMEM((CH,), jnp.int32),
            pltpu.VMEM((CH,), jnp.float32),
            pltpu.VMEM((RPT,), jnp.float32),
            pltpu.VMEM_SHARED((NP,), jnp.float32),
        ],
    )(_sc_deg_body)
    return f(dst)


def _sc_agg_body(g_hbm, src_hbm, dst_hbm, out_hbm,
                 sidxA, didxA, sidxB, didxB,
                 rows0, rows1,
                 acc, isem, gsem0, gsem1):
    cid = lax.axis_index("c")
    sid = lax.axis_index("s")
    gv = g_hbm.at[cid]

    sbuf = (sidxA, sidxB)
    dbuf = (didxA, didxB)
    rbuf = (rows0, rows1)
    gsem = (gsem0, gsem1)

    zeros16 = jnp.zeros((16,), jnp.float32)

    @pl.loop(0, CH)
    def _zrow(r):
        for l in range(D // 16):
            rows0[r, pl.ds(l * 16, 16)] = zeros16

    for k in range(RPT // CH):
        pltpu.sync_copy(rows0, acc.at[pl.ds(sid * RPT + k * CH, CH)])
    plsc.subcore_barrier()

    cbase = sid * NJ

    def load_idx_batch(t, ib):
        row = cbase + t * BT
        pltpu.async_copy(src_hbm.at[pl.ds(row, BT)], sbuf[ib], isem)
        pltpu.async_copy(dst_hbm.at[pl.ds(row, BT)], dbuf[ib], isem)

    def wait_idx_batch(ib):
        pltpu.make_async_copy(src_hbm.at[pl.ds(0, BT)], sbuf[ib], isem).wait()
        pltpu.make_async_copy(dst_hbm.at[pl.ds(0, BT)], dbuf[ib], isem).wait()

    def start_gather(ib, k, b):
        pltpu.async_copy(gv.at[sbuf[ib].at[k]], rbuf[b], gsem[b])

    def wait_gather(ib, k, b):
        pltpu.make_async_copy(gv.at[sbuf[ib].at[k]], rbuf[b], gsem[b]).wait()

    def scatter(ib, k, b):
        pltpu.sync_copy(rbuf[b], acc.at[dbuf[ib].at[k]], add=True)

    load_idx_batch(0, 0)
    wait_idx_batch(0)
    start_gather(0, 0, 0)
    for k in range(BT):
        b = k % NRING
        nb_ = (k + 1) % NRING
        if k == 2:
            load_idx_batch(1, 1)
        if k == BT - 2:
            wait_idx_batch(1)
        if k < BT - 1:
            start_gather(0, k + 1, nb_)
        else:
            start_gather(1, 0, nb_)
        wait_gather(0, k, b)
        scatter(0, k, b)

    @pl.loop(1, NB)
    def _batch(t):
        tb = t % 2
        for parity in range(2):
            @pl.when(tb == parity)
            def _run(parity=parity):
                ib = parity
                nib = 1 - parity
                for k in range(BT):
                    b = k % NRING
                    nb_ = (k + 1) % NRING
                    if k == 2:
                        @pl.when(t + 1 < NB)
                        def _pf():
                            load_idx_batch(t + 1, nib)
                    if k == BT - 2:
                        @pl.when(t + 1 < NB)
                        def _wf():
                            wait_idx_batch(nib)
                    if k < BT - 1:
                        start_gather(ib, k + 1, nb_)
                    else:
                        @pl.when(t + 1 < NB)
                        def _ng():
                            start_gather(nib, 0, nb_)
                    wait_gather(ib, k, b)
                    scatter(ib, k, b)

    plsc.subcore_barrier()
    for k in range(RPT // CH):
        pltpu.sync_copy(acc.at[pl.ds(sid * RPT + k * CH, CH)],
                        out_hbm.at[cid].at[pl.ds(sid * RPT + k * CH, CH)])


def _sc_agg(g, src2d, dst2d):
    f = functools.partial(
        pl.kernel,
        out_type=jax.ShapeDtypeStruct((2, NP, D), jnp.float32),
        mesh=_mesh(),
        scratch_types=[
            pltpu.VMEM((BT, CH), jnp.int32),
            pltpu.VMEM((BT, CH), jnp.int32),
            pltpu.VMEM((BT, CH), jnp.int32),
            pltpu.VMEM((BT, CH), jnp.int32),
            pltpu.VMEM((CH, D), jnp.float32),
            pltpu.VMEM((CH, D), jnp.float32),
            pltpu.VMEM_SHARED((NP, D), jnp.float32),
            pltpu.SemaphoreType.DMA,
            pltpu.SemaphoreType.DMA,
            pltpu.SemaphoreType.DMA,
        ],
    )(_sc_agg_body)
    return f(g, src2d, dst2d)


def _tc1_body(z_ref, x_ref, d2_ref, w_ref, g_ref, dinv_ref):
    deg = d2_ref[0] + d2_ref[1] + 1.0
    dinv = lax.rsqrt(deg)
    dinv_ref[...] = dinv
    g_ref[0] = dinv * jnp.dot(z_ref[...], w_ref[0],
                              preferred_element_type=jnp.float32)
    g_ref[1] = dinv * jnp.dot(x_ref[...], w_ref[1],
                              preferred_element_type=jnp.float32)


def _tc1(z_pad, x_pad, deg2, w1):
    return pl.pallas_call(
        _tc1_body,
        grid=(GRID,),
        in_specs=[
            pl.BlockSpec((BR, D), lambda i: (i, 0)),
            pl.BlockSpec((BR, D), lambda i: (i, 0)),
            pl.BlockSpec((2, BR, 1), lambda i: (0, i, 0)),
            pl.BlockSpec((2, D, D), lambda i: (0, 0, 0)),
        ],
        out_specs=[
            pl.BlockSpec((2, BR, D), lambda i: (0, i, 0)),
            pl.BlockSpec((BR, 1), lambda i: (i, 0)),
        ],
        out_shape=[
            jax.ShapeDtypeStruct((2, NP, D), jnp.float32),
            jax.ShapeDtypeStruct((NP, 1), jnp.float32),
        ],
    )(z_pad, x_pad, deg2, w1)


def _tc2_body(agg_ref, g_ref, dinv_ref, b_ref, w_ref, out_ref):
    dinv = dinv_ref[...]
    for c in range(2):
        u = dinv * (agg_ref[c] + g_ref[c]) + b_ref[c]
        u = jnp.where(u >= 0, u, u * SLOPE)
        out_ref[c] = dinv * jnp.dot(u, w_ref[c],
                                    preferred_element_type=jnp.float32)


def _tc2(agg1, g1, dinv, b1, w2):
    return pl.pallas_call(
        _tc2_body,
        grid=(GRID,),
        in_specs=[
            pl.BlockSpec((2, BR, D), lambda i: (0, i, 0)),
            pl.BlockSpec((2, BR, D), lambda i: (0, i, 0)),
            pl.BlockSpec((BR, 1), lambda i: (i, 0)),
            pl.BlockSpec((2, D), lambda i: (0, 0)),
            pl.BlockSpec((2, D, D), lambda i: (0, 0, 0)),
        ],
        out_specs=pl.BlockSpec((2, BR, D), lambda i: (0, i, 0)),
        out_shape=jax.ShapeDtypeStruct((2, NP, D), jnp.float32),
    )(agg1, g1, dinv, b1, w2)


def _tc3_body(agg_ref, g_ref, dinv_ref, b_ref, wo_ref, bo_ref, out_ref):
    dinv = dinv_ref[...]
    zz = jnp.tanh(dinv * (agg_ref[0] + g_ref[0]) + b_ref[0])
    xx = jnp.tanh(dinv * (agg_ref[1] + g_ref[1]) + b_ref[1])
    out_ref[...] = jnp.dot(zz * xx, wo_ref[...],
                           preferred_element_type=jnp.float32) + bo_ref[...]


def _tc3(agg2, g2, dinv, b2, Wo, bo):
    return pl.pallas_call(
        _tc3_body,
        grid=(GRID,),
        in_specs=[
            pl.BlockSpec((2, BR, D), lambda i: (0, i, 0)),
            pl.BlockSpec((2, BR, D), lambda i: (0, i, 0)),
            pl.BlockSpec((BR, 1), lambda i: (i, 0)),
            pl.BlockSpec((2, D), lambda i: (0, 0)),
            pl.BlockSpec((D, 1), lambda i: (0, 0)),
            pl.BlockSpec((1,), lambda i: (0,)),
        ],
        out_specs=pl.BlockSpec((BR, 1), lambda i: (i, 0)),
        out_shape=jax.ShapeDtypeStruct((NP, 1), jnp.float32),
    )(agg2, g2, dinv, b2, Wo, bo)


@jax.jit
def kernel(z, x, edge_index, We1, be1, We2, be2, Wf1, bf1, Wf2, bf2, Wo, bo):
    pad_idx = N + jnp.arange(EP - E, dtype=jnp.int32) % (NP - N)
    src_p = jnp.concatenate([edge_index[0], pad_idx]).reshape(ECH, CH)
    dst_p = jnp.concatenate([edge_index[1], pad_idx]).reshape(ECH, CH)

    z_pad = jnp.pad(z, ((0, NP - N), (0, 0)))
    x_pad = jnp.pad(x, ((0, NP - N), (0, 0)))
    w1 = jnp.stack([We1, Wf1])
    w2 = jnp.stack([We2, Wf2])
    b1 = jnp.stack([be1, bf1])
    b2 = jnp.stack([be2, bf2])

    deg2 = _sc_deg(dst_p.reshape(EP))
    deg2 = deg2[:, :, None]

    g1, dinv = _tc1(z_pad, x_pad, deg2, w1)
    agg1 = _sc_agg(g1, src_p, dst_p)
    g2 = _tc2(agg1, g1, dinv, b1, w2)
    agg2 = _sc_agg(g2, src_p, dst_p)
    out = _tc3(agg2, g2, dinv, b2, Wo, bo)
    return out[:N]


# speedup vs baseline: 2.7916x; 1.0711x over previous
import functools

import jax
import jax.numpy as jnp
from jax import lax
from jax.experimental import pallas as pl
from jax.experimental.pallas import tpu as pltpu
from jax.experimental.pallas import tpu_sc as plsc

N = 10000
NP = 10240
E = 320000
D = 128
CH = 128
ECH = 2560
EP = ECH * CH
BR = 512
GRID = NP // BR
RPT = NP // 16
SLOPE = (1.0 / 8.0 + 1.0 / 3.0) / 2.0

NJ = ECH // 16
BT = 16
NB = NJ // BT
NRING = 2
NJD = ECH // 32


def _mesh():
    return plsc.VectorSubcoreMesh(core_axis_name="c", subcore_axis_name="s")


NBD = NJD // BT


def _sc_deg_body(dst_hbm, out_hbm, didxA, didxB, ones_v, zbuf, acc, isem):
    cid = lax.axis_index("c")
    sid = lax.axis_index("s")
    wid = cid * 16 + sid
    dbuf = (didxA, didxB)

    for l in range(8):
        ones_v[pl.ds(l * 16, 16)] = jnp.full((16,), 1.0, jnp.float32)
    zeros16 = jnp.zeros((16,), jnp.float32)

    @pl.loop(0, RPT // 16)
    def _zero(i):
        zbuf[pl.ds(i * 16, 16)] = zeros16

    pltpu.sync_copy(zbuf, acc.at[pl.ds(sid * RPT, RPT)])
    plsc.subcore_barrier()

    cbase = wid * NJD

    def load_idx(t, ib):
        pltpu.async_copy(dst_hbm.at[pl.ds(cbase + t * BT, BT)],
                         dbuf[ib], isem)

    def wait_idx(ib):
        pltpu.make_async_copy(dst_hbm.at[pl.ds(0, BT)], dbuf[ib], isem).wait()

    load_idx(0, 0)
    wait_idx(0)
    load_idx(1, 1)
    for k in range(BT):
        pltpu.sync_copy(ones_v, acc.at[dbuf[0].at[k]], add=True)

    @pl.loop(1, NBD)
    def _batch(t):
        tb = t % 2
        for parity in range(2):
            @pl.when(tb == parity)
            def _run(parity=parity):
                ib = parity
                nib = 1 - parity
                for k in range(BT):
                    if k == 0:
                        wait_idx(ib)
                    if k == 2:
                        @pl.when(t + 1 < NBD)
                        def _pf():
                            load_idx(t + 1, nib)
                    pltpu.sync_copy(ones_v, acc.at[dbuf[ib].at[k]], add=True)

    plsc.subcore_barrier()
    pltpu.sync_copy(acc.at[pl.ds(sid * RPT, RPT)],
                    out_hbm.at[cid, pl.ds(sid * RPT, RPT)])


def _sc_deg(dst2d):
    f = functools.partial(
        pl.kernel,
        out_type=jax.ShapeDtypeStruct((2, NP), jnp.float32),
        mesh=_mesh(),
        scratch_types=[
            pltpu.VMEM((BT, CH), jnp.int32),
            pltpu.VMEM((BT, CH), jnp.int32),
            pltpu.VMEM((CH,), jnp.float32),
            pltpu.VMEM((RPT,), jnp.float32),
            pltpu.VMEM_SHARED((NP,), jnp.float32),
            pltpu.SemaphoreType.DMA,
        ],
    )(_sc_deg_body)
    return f(dst2d)


def _sc_agg_body(g_hbm, src_hbm, dst_hbm, out_hbm,
                 sidxA, didxA, sidxB, didxB,
                 rows0, rows1,
                 acc, isem, gsem0, gsem1):
    cid = lax.axis_index("c")
    sid = lax.axis_index("s")
    gv = g_hbm.at[cid]

    sbuf = (sidxA, sidxB)
    dbuf = (didxA, didxB)
    rbuf = (rows0, rows1)
    gsem = (gsem0, gsem1)

    zeros16 = jnp.zeros((16,), jnp.float32)

    @pl.loop(0, CH)
    def _zrow(r):
        for l in range(D // 16):
            rows0[r, pl.ds(l * 16, 16)] = zeros16

    for k in range(RPT // CH):
        pltpu.sync_copy(rows0, acc.at[pl.ds(sid * RPT + k * CH, CH)])
    plsc.subcore_barrier()

    cbase = sid * NJ

    def load_idx_batch(t, ib):
        row = cbase + t * BT
        pltpu.async_copy(src_hbm.at[pl.ds(row, BT)], sbuf[ib], isem)
        pltpu.async_copy(dst_hbm.at[pl.ds(row, BT)], dbuf[ib], isem)

    def wait_idx_batch(ib):
        pltpu.make_async_copy(src_hbm.at[pl.ds(0, BT)], sbuf[ib], isem).wait()
        pltpu.make_async_copy(dst_hbm.at[pl.ds(0, BT)], dbuf[ib], isem).wait()

    def start_gather(ib, k, b):
        pltpu.async_copy(gv.at[sbuf[ib].at[k]], rbuf[b], gsem[b])

    def wait_gather(ib, k, b):
        pltpu.make_async_copy(gv.at[sbuf[ib].at[k]], rbuf[b], gsem[b]).wait()

    def scatter(ib, k, b):
        pltpu.sync_copy(rbuf[b], acc.at[dbuf[ib].at[k]], add=True)

    load_idx_batch(0, 0)
    wait_idx_batch(0)
    start_gather(0, 0, 0)
    for k in range(BT):
        b = k % NRING
        nb_ = (k + 1) % NRING
        if k == 2:
            load_idx_batch(1, 1)
        if k == BT - 2:
            wait_idx_batch(1)
        if k < BT - 1:
            start_gather(0, k + 1, nb_)
        else:
            start_gather(1, 0, nb_)
        wait_gather(0, k, b)
        scatter(0, k, b)

    @pl.loop(1, NB)
    def _batch(t):
        tb = t % 2
        for parity in range(2):
            @pl.when(tb == parity)
            def _run(parity=parity):
                ib = parity
                nib = 1 - parity
                for k in range(BT):
                    b = k % NRING
                    nb_ = (k + 1) % NRING
                    if k == 2:
                        @pl.when(t + 1 < NB)
                        def _pf():
                            load_idx_batch(t + 1, nib)
                    if k == BT - 2:
                        @pl.when(t + 1 < NB)
                        def _wf():
                            wait_idx_batch(nib)
                    if k < BT - 1:
                        start_gather(ib, k + 1, nb_)
                    else:
                        @pl.when(t + 1 < NB)
                        def _ng():
                            start_gather(nib, 0, nb_)
                    wait_gather(ib, k, b)
                    scatter(ib, k, b)

    plsc.subcore_barrier()
    for k in range(RPT // CH):
        pltpu.sync_copy(acc.at[pl.ds(sid * RPT + k * CH, CH)],
                        out_hbm.at[cid].at[pl.ds(sid * RPT + k * CH, CH)])


def _sc_agg(g, src2d, dst2d):
    f = functools.partial(
        pl.kernel,
        out_type=jax.ShapeDtypeStruct((2, NP, D), jnp.float32),
        mesh=_mesh(),
        scratch_types=[
            pltpu.VMEM((BT, CH), jnp.int32),
            pltpu.VMEM((BT, CH), jnp.int32),
            pltpu.VMEM((BT, CH), jnp.int32),
            pltpu.VMEM((BT, CH), jnp.int32),
            pltpu.VMEM((CH, D), jnp.float32),
            pltpu.VMEM((CH, D), jnp.float32),
            pltpu.VMEM_SHARED((NP, D), jnp.float32),
            pltpu.SemaphoreType.DMA,
            pltpu.SemaphoreType.DMA,
            pltpu.SemaphoreType.DMA,
        ],
    )(_sc_agg_body)
    return f(g, src2d, dst2d)


def _tc1_body(z_ref, x_ref, d2_ref, w_ref, g_ref, dinv_ref):
    deg = d2_ref[0] + d2_ref[1] + 1.0
    dinv = lax.rsqrt(deg)
    dinv_ref[...] = dinv
    g_ref[0] = dinv * jnp.dot(z_ref[...], w_ref[0],
                              preferred_element_type=jnp.float32)
    g_ref[1] = dinv * jnp.dot(x_ref[...], w_ref[1],
                              preferred_element_type=jnp.float32)


def _tc1(z_pad, x_pad, deg2, w1):
    return pl.pallas_call(
        _tc1_body,
        grid=(GRID,),
        in_specs=[
            pl.BlockSpec((BR, D), lambda i: (i, 0)),
            pl.BlockSpec((BR, D), lambda i: (i, 0)),
            pl.BlockSpec((2, BR, 1), lambda i: (0, i, 0)),
            pl.BlockSpec((2, D, D), lambda i: (0, 0, 0)),
        ],
        out_specs=[
            pl.BlockSpec((2, BR, D), lambda i: (0, i, 0)),
            pl.BlockSpec((BR, 1), lambda i: (i, 0)),
        ],
        out_shape=[
            jax.ShapeDtypeStruct((2, NP, D), jnp.float32),
            jax.ShapeDtypeStruct((NP, 1), jnp.float32),
        ],
    )(z_pad, x_pad, deg2, w1)


def _tc2_body(agg_ref, g_ref, dinv_ref, b_ref, w_ref, out_ref):
    dinv = dinv_ref[...]
    for c in range(2):
        u = dinv * (agg_ref[c] + g_ref[c]) + b_ref[c]
        u = jnp.where(u >= 0, u, u * SLOPE)
        out_ref[c] = dinv * jnp.dot(u, w_ref[c],
                                    preferred_element_type=jnp.float32)


def _tc2(agg1, g1, dinv, b1, w2):
    return pl.pallas_call(
        _tc2_body,
        grid=(GRID,),
        in_specs=[
            pl.BlockSpec((2, BR, D), lambda i: (0, i, 0)),
            pl.BlockSpec((2, BR, D), lambda i: (0, i, 0)),
            pl.BlockSpec((BR, 1), lambda i: (i, 0)),
            pl.BlockSpec((2, D), lambda i: (0, 0)),
            pl.BlockSpec((2, D, D), lambda i: (0, 0, 0)),
        ],
        out_specs=pl.BlockSpec((2, BR, D), lambda i: (0, i, 0)),
        out_shape=jax.ShapeDtypeStruct((2, NP, D), jnp.float32),
    )(agg1, g1, dinv, b1, w2)


def _tc3_body(agg_ref, g_ref, dinv_ref, b_ref, wo_ref, bo_ref, out_ref):
    dinv = dinv_ref[...]
    zz = jnp.tanh(dinv * (agg_ref[0] + g_ref[0]) + b_ref[0])
    xx = jnp.tanh(dinv * (agg_ref[1] + g_ref[1]) + b_ref[1])
    out_ref[...] = jnp.dot(zz * xx, wo_ref[...],
                           preferred_element_type=jnp.float32) + bo_ref[...]


def _tc3(agg2, g2, dinv, b2, Wo, bo):
    return pl.pallas_call(
        _tc3_body,
        grid=(GRID,),
        in_specs=[
            pl.BlockSpec((2, BR, D), lambda i: (0, i, 0)),
            pl.BlockSpec((2, BR, D), lambda i: (0, i, 0)),
            pl.BlockSpec((BR, 1), lambda i: (i, 0)),
            pl.BlockSpec((2, D), lambda i: (0, 0)),
            pl.BlockSpec((D, 1), lambda i: (0, 0)),
            pl.BlockSpec((1,), lambda i: (0,)),
        ],
        out_specs=pl.BlockSpec((BR, 1), lambda i: (i, 0)),
        out_shape=jax.ShapeDtypeStruct((NP, 1), jnp.float32),
    )(agg2, g2, dinv, b2, Wo, bo)


@jax.jit
def kernel(z, x, edge_index, We1, be1, We2, be2, Wf1, bf1, Wf2, bf2, Wo, bo):
    pad_idx = N + jnp.arange(EP - E, dtype=jnp.int32) % (NP - N)
    src_p = jnp.concatenate([edge_index[0], pad_idx]).reshape(ECH, CH)
    dst_p = jnp.concatenate([edge_index[1], pad_idx]).reshape(ECH, CH)

    z_pad = jnp.pad(z, ((0, NP - N), (0, 0)))
    x_pad = jnp.pad(x, ((0, NP - N), (0, 0)))
    w1 = jnp.stack([We1, Wf1])
    w2 = jnp.stack([We2, Wf2])
    b1 = jnp.stack([be1, bf1])
    b2 = jnp.stack([be2, bf2])

    deg2 = _sc_deg(dst_p)
    deg2 = deg2[:, :, None]

    g1, dinv = _tc1(z_pad, x_pad, deg2, w1)
    agg1 = _sc_agg(g1, src_p, dst_p)
    g2 = _tc2(agg1, g1, dinv, b1, w2)
    agg2 = _sc_agg(g2, src_p, dst_p)
    out = _tc3(agg2, g2, dinv, b2, Wo, bo)
    return out[:N]


# speedup vs baseline: 2.8005x; 1.0032x over previous
import functools

import jax
import jax.numpy as jnp
from jax import lax
from jax.experimental import pallas as pl
from jax.experimental.pallas import tpu as pltpu
from jax.experimental.pallas import tpu_sc as plsc

N = 10000
NP = 10240
E = 320000
D = 128
CH = 128
ECH = 2560
EP = ECH * CH
BR = 512
GRID = NP // BR
RPT = NP // 16
SLOPE = (1.0 / 8.0 + 1.0 / 3.0) / 2.0

NJ = ECH // 16
BT = 16
NB = NJ // BT
NRING = 2
NJD = ECH // 32


def _mesh():
    return plsc.VectorSubcoreMesh(core_axis_name="c", subcore_axis_name="s")


NBD = NJD // BT


def _sc_deg_body(dst_hbm, out_hbm, didxA, didxB, ones_v, zbuf, acc, isem):
    cid = lax.axis_index("c")
    sid = lax.axis_index("s")
    wid = cid * 16 + sid
    dbuf = (didxA, didxB)

    for l in range(8):
        ones_v[pl.ds(l * 16, 16)] = jnp.full((16,), 1.0, jnp.float32)
    zeros16 = jnp.zeros((16,), jnp.float32)

    @pl.loop(0, RPT // 16)
    def _zero(i):
        zbuf[pl.ds(i * 16, 16)] = zeros16

    pltpu.sync_copy(zbuf, acc.at[pl.ds(sid * RPT, RPT)])
    plsc.subcore_barrier()

    cbase = wid * NJD

    def load_idx(t, ib):
        pltpu.async_copy(dst_hbm.at[pl.ds(cbase + t * BT, BT)],
                         dbuf[ib], isem)

    def wait_idx(ib):
        pltpu.make_async_copy(dst_hbm.at[pl.ds(0, BT)], dbuf[ib], isem).wait()

    load_idx(0, 0)
    wait_idx(0)
    load_idx(1, 1)
    for k in range(BT):
        pltpu.sync_copy(ones_v, acc.at[dbuf[0].at[k]], add=True)

    @pl.loop(1, NBD)
    def _batch(t):
        tb = t % 2
        for parity in range(2):
            @pl.when(tb == parity)
            def _run(parity=parity):
                ib = parity
                nib = 1 - parity
                for k in range(BT):
                    if k == 0:
                        wait_idx(ib)
                    if k == 2:
                        @pl.when(t + 1 < NBD)
                        def _pf():
                            load_idx(t + 1, nib)
                    pltpu.sync_copy(ones_v, acc.at[dbuf[ib].at[k]], add=True)

    plsc.subcore_barrier()
    pltpu.sync_copy(acc.at[pl.ds(sid * RPT, RPT)],
                    out_hbm.at[cid, pl.ds(sid * RPT, RPT)])


def _sc_deg(dst2d):
    f = functools.partial(
        pl.kernel,
        out_type=jax.ShapeDtypeStruct((2, NP), jnp.float32),
        mesh=_mesh(),
        scratch_types=[
            pltpu.VMEM((BT, CH), jnp.int32),
            pltpu.VMEM((BT, CH), jnp.int32),
            pltpu.VMEM((CH,), jnp.float32),
            pltpu.VMEM((RPT,), jnp.float32),
            pltpu.VMEM_SHARED((NP,), jnp.float32),
            pltpu.SemaphoreType.DMA,
        ],
    )(_sc_deg_body)
    return f(dst2d)


def _sc_agg_body(g_hbm, src_hbm, dst_hbm, out_hbm,
                 sidxA, didxA, sidxB, didxB,
                 rows0, rows1,
                 acc, isem, gsem0, gsem1):
    cid = lax.axis_index("c")
    sid = lax.axis_index("s")
    gv = g_hbm.at[cid]

    sbuf = (sidxA, sidxB)
    dbuf = (didxA, didxB)
    rbuf = (rows0, rows1)
    gsem = (gsem0, gsem1)

    zeros16 = jnp.zeros((16,), jnp.float32)

    @pl.loop(0, CH)
    def _zrow(r):
        for l in range(D // 16):
            rows0[r, pl.ds(l * 16, 16)] = zeros16

    for k in range(RPT // CH):
        pltpu.sync_copy(rows0, acc.at[pl.ds(sid * RPT + k * CH, CH)])
    plsc.subcore_barrier()

    cbase = sid * NJ

    def load_idx_batch(t, ib):
        row = cbase + t * BT
        pltpu.async_copy(src_hbm.at[pl.ds(row, BT)], sbuf[ib], isem)
        pltpu.async_copy(dst_hbm.at[pl.ds(row, BT)], dbuf[ib], isem)

    def wait_idx_batch(ib):
        pltpu.make_async_copy(src_hbm.at[pl.ds(0, BT)], sbuf[ib], isem).wait()
        pltpu.make_async_copy(dst_hbm.at[pl.ds(0, BT)], dbuf[ib], isem).wait()

    def start_gather(ib, k, b):
        pltpu.async_copy(gv.at[sbuf[ib].at[k]], rbuf[b], gsem[b])

    def wait_gather(ib, k, b):
        pltpu.make_async_copy(gv.at[sbuf[ib].at[k]], rbuf[b], gsem[b]).wait()

    def scatter(ib, k, b):
        pltpu.sync_copy(rbuf[b], acc.at[dbuf[ib].at[k]], add=True)

    load_idx_batch(0, 0)
    wait_idx_batch(0)
    start_gather(0, 0, 0)
    for k in range(BT):
        b = k % NRING
        nb_ = (k + 1) % NRING
        if k == 2:
            load_idx_batch(1, 1)
        if k == BT - 2:
            wait_idx_batch(1)
        if k < BT - 1:
            start_gather(0, k + 1, nb_)
        else:
            start_gather(1, 0, nb_)
        wait_gather(0, k, b)
        scatter(0, k, b)

    @pl.loop(1, NB)
    def _batch(t):
        tb = t % 2
        for parity in range(2):
            @pl.when(tb == parity)
            def _run(parity=parity):
                ib = parity
                nib = 1 - parity
                for k in range(BT):
                    b = k % NRING
                    nb_ = (k + 1) % NRING
                    if k == 2:
                        @pl.when(t + 1 < NB)
                        def _pf():
                            load_idx_batch(t + 1, nib)
                    if k == BT - 2:
                        @pl.when(t + 1 < NB)
                        def _wf():
                            wait_idx_batch(nib)
                    if k < BT - 1:
                        start_gather(ib, k + 1, nb_)
                    else:
                        @pl.when(t + 1 < NB)
                        def _ng():
                            start_gather(nib, 0, nb_)
                    wait_gather(ib, k, b)
                    scatter(ib, k, b)

    plsc.subcore_barrier()
    for k in range(RPT // CH):
        pltpu.sync_copy(acc.at[pl.ds(sid * RPT + k * CH, CH)],
                        out_hbm.at[cid].at[pl.ds(sid * RPT + k * CH, CH)])


def _sc_agg(g, src2d, dst2d):
    f = functools.partial(
        pl.kernel,
        out_type=jax.ShapeDtypeStruct((2, NP, D), jnp.float32),
        mesh=_mesh(),
        scratch_types=[
            pltpu.VMEM((BT, CH), jnp.int32),
            pltpu.VMEM((BT, CH), jnp.int32),
            pltpu.VMEM((BT, CH), jnp.int32),
            pltpu.VMEM((BT, CH), jnp.int32),
            pltpu.VMEM((CH, D), jnp.float32),
            pltpu.VMEM((CH, D), jnp.float32),
            pltpu.VMEM_SHARED((NP, D), jnp.float32),
            pltpu.SemaphoreType.DMA,
            pltpu.SemaphoreType.DMA,
            pltpu.SemaphoreType.DMA,
        ],
    )(_sc_agg_body)
    return f(g, src2d, dst2d)


def _tc1_body(z_ref, x_ref, d2_ref, w_ref, g_ref, dinv_ref):
    deg = d2_ref[0] + d2_ref[1] + 1.0
    dinv = lax.rsqrt(deg)
    dinv_ref[...] = dinv
    g_ref[0] = dinv * jnp.dot(z_ref[...], w_ref[0],
                              preferred_element_type=jnp.float32)
    g_ref[1] = dinv * jnp.dot(x_ref[...], w_ref[1],
                              preferred_element_type=jnp.float32)


def _tc1(z_in, x_in, deg2, w1):
    return pl.pallas_call(
        _tc1_body,
        grid=(GRID,),
        in_specs=[
            pl.BlockSpec((BR, D), lambda i: (i, 0)),
            pl.BlockSpec((BR, D), lambda i: (i, 0)),
            pl.BlockSpec((2, BR, 1), lambda i: (0, i, 0)),
            pl.BlockSpec((2, D, D), lambda i: (0, 0, 0)),
        ],
        out_specs=[
            pl.BlockSpec((2, BR, D), lambda i: (0, i, 0)),
            pl.BlockSpec((BR, 1), lambda i: (i, 0)),
        ],
        out_shape=[
            jax.ShapeDtypeStruct((2, NP, D), jnp.float32),
            jax.ShapeDtypeStruct((NP, 1), jnp.float32),
        ],
    )(z_in, x_in, deg2, w1)


def _tc2_body(agg_ref, g_ref, dinv_ref, b_ref, w_ref, out_ref):
    dinv = dinv_ref[...]
    for c in range(2):
        u = dinv * (agg_ref[c] + g_ref[c]) + b_ref[c]
        u = jnp.where(u >= 0, u, u * SLOPE)
        out_ref[c] = dinv * jnp.dot(u, w_ref[c],
                                    preferred_element_type=jnp.float32)


def _tc2(agg1, g1, dinv, b1, w2):
    return pl.pallas_call(
        _tc2_body,
        grid=(GRID,),
        in_specs=[
            pl.BlockSpec((2, BR, D), lambda i: (0, i, 0)),
            pl.BlockSpec((2, BR, D), lambda i: (0, i, 0)),
            pl.BlockSpec((BR, 1), lambda i: (i, 0)),
            pl.BlockSpec((2, D), lambda i: (0, 0)),
            pl.BlockSpec((2, D, D), lambda i: (0, 0, 0)),
        ],
        out_specs=pl.BlockSpec((2, BR, D), lambda i: (0, i, 0)),
        out_shape=jax.ShapeDtypeStruct((2, NP, D), jnp.float32),
    )(agg1, g1, dinv, b1, w2)


def _tc3_body(agg_ref, g_ref, dinv_ref, b_ref, wo_ref, bo_ref, out_ref):
    dinv = dinv_ref[...]
    zz = jnp.tanh(dinv * (agg_ref[0] + g_ref[0]) + b_ref[0])
    xx = jnp.tanh(dinv * (agg_ref[1] + g_ref[1]) + b_ref[1])
    out_ref[...] = jnp.dot(zz * xx, wo_ref[...],
                           preferred_element_type=jnp.float32) + bo_ref[...]


def _tc3(agg2, g2, dinv, b2, Wo, bo):
    return pl.pallas_call(
        _tc3_body,
        grid=(GRID,),
        in_specs=[
            pl.BlockSpec((2, BR, D), lambda i: (0, i, 0)),
            pl.BlockSpec((2, BR, D), lambda i: (0, i, 0)),
            pl.BlockSpec((BR, 1), lambda i: (i, 0)),
            pl.BlockSpec((2, D), lambda i: (0, 0)),
            pl.BlockSpec((D, 1), lambda i: (0, 0)),
            pl.BlockSpec((1,), lambda i: (0,)),
        ],
        out_specs=pl.BlockSpec((BR, 1), lambda i: (i, 0)),
        out_shape=jax.ShapeDtypeStruct((NP, 1), jnp.float32),
    )(agg2, g2, dinv, b2, Wo, bo)


@jax.jit
def kernel(z, x, edge_index, We1, be1, We2, be2, Wf1, bf1, Wf2, bf2, Wo, bo):
    pad_idx = N + jnp.arange(EP - E, dtype=jnp.int32) % (NP - N)
    src_p = jnp.concatenate([edge_index[0], pad_idx]).reshape(ECH, CH)
    dst_p = jnp.concatenate([edge_index[1], pad_idx]).reshape(ECH, CH)

    w1 = jnp.stack([We1, Wf1])
    w2 = jnp.stack([We2, Wf2])
    b1 = jnp.stack([be1, bf1])
    b2 = jnp.stack([be2, bf2])

    deg2 = _sc_deg(dst_p)
    deg2 = deg2[:, :, None]

    g1, dinv = _tc1(z, x, deg2, w1)
    agg1 = _sc_agg(g1, src_p, dst_p)
    g2 = _tc2(agg1, g1, dinv, b1, w2)
    agg2 = _sc_agg(g2, src_p, dst_p)
    out = _tc3(agg2, g2, dinv, b2, Wo, bo)
    return out[:N]
